# Initial kernel scaffold; baseline (speedup 1.0000x reference)
import jax
import jax.numpy as jnp
from jax.experimental import pallas as pl


def kernel(inputs, embedding, ema_w, ema_cluster_size):
    raise NotImplementedError("write your pallas kernel here")


# speedup vs baseline: 2.1941x; 2.1941x over previous
import functools
import jax
import jax.numpy as jnp
from jax import lax
from jax.experimental import pallas as pl
from jax.experimental.pallas import tpu as pltpu

K = 512
D = 32
DECAY = 0.9
EPS = 1e-5


def _body(T, NB, N, x_ref, embT_ref, emawT_ref, cs_ref, out_ref,
          counts_ref, dwT_ref, sumx2_ref):
    i = pl.program_id(0)

    @pl.when(i == 0)
    def _():
        counts_ref[...] = jnp.zeros_like(counts_ref)
        dwT_ref[...] = jnp.zeros_like(dwT_ref)
        sumx2_ref[0] = 0.0

    x = x_ref[...]
    embT = embT_ref[...]
    scores = jnp.dot(x, embT, preferred_element_type=jnp.float32)
    e2 = jnp.sum(embT * embT, axis=0, keepdims=True)
    dist = e2 - 2.0 * scores
    mind = jnp.min(dist, axis=1, keepdims=True)
    iota = lax.broadcasted_iota(jnp.int32, (T, K), 1)
    idx = jnp.min(jnp.where(dist == mind, iota, K), axis=1, keepdims=True)
    one_hot = (iota == idx).astype(jnp.float32)
    counts_ref[...] += jnp.sum(one_hot, axis=0, keepdims=True)
    dwT_ref[...] += lax.dot_general(
        x, one_hot, (((0,), (0,)), ((), ())),
        preferred_element_type=jnp.float32)
    sumx2_ref[0] += jnp.sum(x * x)

    @pl.when(i == NB - 1)
    def _():
        counts = counts_ref[...]
        cs = cs_ref[...] * DECAY + (1.0 - DECAY) * counts
        n = jnp.sum(cs)
        csn = (cs + EPS) / (n + K * EPS) * n
        ema_w_new = emawT_ref[...] * DECAY + (1.0 - DECAY) * dwT_ref[...]
        e_new = ema_w_new / csn
        s1 = jnp.sum(dwT_ref[...] * e_new)
        s2 = jnp.sum(counts * jnp.sum(e_new * e_new, axis=0, keepdims=True))
        loss = (sumx2_ref[0] - 2.0 * s1 + s2) / (N * D)
        out_ref[...] = jnp.reshape(loss, (1, 1))


def kernel(inputs, embedding, ema_w, ema_cluster_size):
    N = inputs.shape[0] * inputs.shape[1]
    T = 4096
    NB = N // T
    flat = inputs.reshape(N, D)
    embT = embedding.T
    emawT = ema_w.T
    cs = ema_cluster_size.reshape(1, K)
    out = pl.pallas_call(
        functools.partial(_body, T, NB, N),
        grid=(NB,),
        in_specs=[
            pl.BlockSpec((T, D), lambda i: (i, 0)),
            pl.BlockSpec((D, K), lambda i: (0, 0)),
            pl.BlockSpec((D, K), lambda i: (0, 0)),
            pl.BlockSpec((1, K), lambda i: (0, 0)),
        ],
        out_specs=pl.BlockSpec((1, 1), lambda i: (0, 0)),
        out_shape=jax.ShapeDtypeStruct((1, 1), jnp.float32),
        scratch_shapes=[
            pltpu.VMEM((1, K), jnp.float32),
            pltpu.VMEM((D, K), jnp.float32),
            pltpu.SMEM((1,), jnp.float32),
        ],
    )(flat, embT, emawT, cs)
    return out[0, 0]


# speedup vs baseline: 3.2736x; 1.4920x over previous
import functools
import jax
import jax.numpy as jnp
from jax import lax
from jax.experimental import pallas as pl
from jax.experimental.pallas import tpu as pltpu

K = 512
D = 32
DA = 40
DECAY = 0.9
EPS = 1e-5


def _body(T, NB, N, x_ref, embT_ref, emawT_ref, cs_ref, out_ref,
          embA_ref, dwA_ref, sumx2_ref):
    i = pl.program_id(0)

    @pl.when(i == 0)
    def _():
        embT = embT_ref[...]
        e2 = jnp.sum(embT * embT, axis=0, keepdims=True)
        embA_ref[...] = jnp.concatenate(
            [-2.0 * embT, e2, jnp.zeros((DA - D - 1, K), jnp.float32)], axis=0)
        dwA_ref[...] = jnp.zeros_like(dwA_ref)
        sumx2_ref[0] = 0.0

    x = x_ref[...]
    xa = jnp.concatenate([x, jnp.ones((T, DA - D), jnp.float32)], axis=1)
    dist = jnp.dot(xa, embA_ref[...], preferred_element_type=jnp.float32)
    mind = jnp.min(dist, axis=1, keepdims=True)
    fiota = lax.broadcasted_iota(jnp.int32, (T, K), 1).astype(jnp.float32)
    cand = jnp.where(dist == mind, fiota, float(K))
    idx = jnp.min(cand, axis=1, keepdims=True)
    one_hot = jnp.where(cand == idx, 1.0, 0.0)
    dwA_ref[...] += lax.dot_general(
        xa, one_hot, (((0,), (0,)), ((), ())),
        preferred_element_type=jnp.float32)
    sumx2_ref[0] += jnp.sum(x * x)

    @pl.when(i == NB - 1)
    def _():
        counts = dwA_ref[D:D + 1, :]
        dwT = dwA_ref[0:D, :]
        cs = cs_ref[...] * DECAY + (1.0 - DECAY) * counts
        n = jnp.sum(cs)
        csn = (cs + EPS) / (n + K * EPS) * n
        ema_w_new = emawT_ref[...] * DECAY + (1.0 - DECAY) * dwT
        e_new = ema_w_new / csn
        s1 = jnp.sum(dwT * e_new)
        s2 = jnp.sum(counts * jnp.sum(e_new * e_new, axis=0, keepdims=True))
        loss = (sumx2_ref[0] - 2.0 * s1 + s2) / (N * D)
        out_ref[...] = jnp.reshape(loss, (1, 1))


def kernel(inputs, embedding, ema_w, ema_cluster_size):
    N = inputs.shape[0] * inputs.shape[1]
    T = 4096
    NB = N // T
    flat = inputs.reshape(N, D)
    embT = embedding.T
    emawT = ema_w.T
    cs = ema_cluster_size.reshape(1, K)
    out = pl.pallas_call(
        functools.partial(_body, T, NB, N),
        grid=(NB,),
        in_specs=[
            pl.BlockSpec((T, D), lambda i: (i, 0)),
            pl.BlockSpec((D, K), lambda i: (0, 0)),
            pl.BlockSpec((D, K), lambda i: (0, 0)),
            pl.BlockSpec((1, K), lambda i: (0, 0)),
        ],
        out_specs=pl.BlockSpec((1, 1), lambda i: (0, 0)),
        out_shape=jax.ShapeDtypeStruct((1, 1), jnp.float32),
        scratch_shapes=[
            pltpu.VMEM((DA, K), jnp.float32),
            pltpu.VMEM((DA, K), jnp.float32),
            pltpu.SMEM((1,), jnp.float32),
        ],
    )(flat, embT, emawT, cs)
    return out[0, 0]


# speedup vs baseline: 3.3789x; 1.0321x over previous
import functools
import jax
import jax.numpy as jnp
from jax import lax
from jax.experimental import pallas as pl
from jax.experimental.pallas import tpu as pltpu

K = 512
D = 32
DA = 40
DECAY = 0.9
EPS = 1e-5


def _body(T, NB, N, x_ref, embT_ref, emawT_ref, cs_ref, out_ref,
          embA_ref, dwA_ref, sumx2_ref):
    i = pl.program_id(0)

    @pl.when(i == 0)
    def _():
        embT = embT_ref[...]
        e2 = jnp.sum(embT * embT, axis=0, keepdims=True)
        embA_ref[...] = jnp.concatenate(
            [-2.0 * embT, e2, jnp.zeros((DA - D - 1, K), jnp.float32)], axis=0)
        dwA_ref[...] = jnp.zeros_like(dwA_ref)
        sumx2_ref[0] = 0.0

    x = x_ref[...]
    xa = jnp.concatenate([x, jnp.ones((T, DA - D), jnp.float32)], axis=1)
    dist = jnp.dot(xa, embA_ref[...], preferred_element_type=jnp.float32)
    mind = jnp.min(dist, axis=1, keepdims=True)
    fiota = lax.broadcasted_iota(jnp.int32, (T, K), 1).astype(jnp.float32)
    cand = jnp.where(dist == mind, fiota, float(K))
    idx = jnp.min(cand, axis=1, keepdims=True)
    one_hot = jnp.where(cand == idx, 1.0, 0.0)
    dwA_ref[...] += lax.dot_general(
        xa, one_hot, (((0,), (0,)), ((), ())),
        preferred_element_type=jnp.float32)
    sumx2_ref[0] += jnp.sum(x * x)

    @pl.when(i == NB - 1)
    def _():
        counts = dwA_ref[D:D + 1, :]
        dwT = dwA_ref[0:D, :]
        cs = cs_ref[...] * DECAY + (1.0 - DECAY) * counts
        n = jnp.sum(cs)
        csn = (cs + EPS) / (n + K * EPS) * n
        ema_w_new = emawT_ref[...] * DECAY + (1.0 - DECAY) * dwT
        e_new = ema_w_new / csn
        s1 = jnp.sum(dwT * e_new)
        s2 = jnp.sum(counts * jnp.sum(e_new * e_new, axis=0, keepdims=True))
        loss = (sumx2_ref[0] - 2.0 * s1 + s2) / (N * D)
        out_ref[...] = jnp.reshape(loss, (1, 1))


def kernel(inputs, embedding, ema_w, ema_cluster_size):
    N = inputs.shape[0] * inputs.shape[1]
    T = 8192
    NB = N // T
    flat = inputs.reshape(N, D)
    embT = embedding.T
    emawT = ema_w.T
    cs = ema_cluster_size.reshape(1, K)
    out = pl.pallas_call(
        functools.partial(_body, T, NB, N),
        grid=(NB,),
        in_specs=[
            pl.BlockSpec((T, D), lambda i: (i, 0)),
            pl.BlockSpec((D, K), lambda i: (0, 0)),
            pl.BlockSpec((D, K), lambda i: (0, 0)),
            pl.BlockSpec((1, K), lambda i: (0, 0)),
        ],
        out_specs=pl.BlockSpec((1, 1), lambda i: (0, 0)),
        out_shape=jax.ShapeDtypeStruct((1, 1), jnp.float32),
        scratch_shapes=[
            pltpu.VMEM((DA, K), jnp.float32),
            pltpu.VMEM((DA, K), jnp.float32),
            pltpu.SMEM((1,), jnp.float32),
        ],
    )(flat, embT, emawT, cs)
    return out[0, 0]
